# baseline (speedup 1.0000x reference)
import functools

import jax
import jax.numpy as jnp
from jax import lax
from jax.experimental import pallas as pl
from jax.experimental.pallas import tpu as pltpu
from jax.experimental.pallas import tpu_sc as plsc

N_NODES = 10000
N_EDGES = 320000
D = 128
ED = 16

_NODE_BLK = 2000
_EDGE_BLK = 2000
_CH = 80


def _proj_body(h_ref, w_ref, b_ref, t_ref):
    t_ref[...] = lax.dot_general(h_ref[...], w_ref[0],
                                 (((1,), (1,)), ((), ())),
                                 preferred_element_type=jnp.float32) + b_ref[0]


def _msg_body(spk_ref, ee_ref, eo_ref, w1cp_ref, w2p_ref, b2_ref,
              me_ref, mo_ref):
    w = spk_ref[...]
    lo = lax.bitcast_convert_type(lax.shift_left(w, 16), jnp.float32)
    hi = lax.bitcast_convert_type(
        jnp.bitwise_and(w, jnp.int32(-65536)), jnp.float32)
    even_pre = jnp.concatenate([lo[:, :D // 2], hi[:, :D // 2]], axis=1)
    odd_pre = jnp.concatenate([lo[:, D // 2:], hi[:, D // 2:]], axis=1)
    dn = (((1,), (1,)), ((), ()))
    ce = lax.dot_general(ee_ref[...], w1cp_ref[...], dn,
                         preferred_element_type=jnp.float32)
    co = lax.dot_general(eo_ref[...], w1cp_ref[...], dn,
                         preferred_element_type=jnp.float32)
    he = jnp.maximum(even_pre + ce, 0.0)
    ho = jnp.maximum(odd_pre + co, 0.0)
    me_ref[...] = lax.dot_general(he, w2p_ref[...], dn,
                                  preferred_element_type=jnp.float32) + b2_ref[...]
    mo_ref[...] = lax.dot_general(ho, w2p_ref[...], dn,
                                  preferred_element_type=jnp.float32) + b2_ref[...]


def _update_body(h_ref, p_ref, w3a_ref, w3b_ref, b3_ref, out_ref):
    agg = p_ref[0] + p_ref[1]
    pre = (lax.dot_general(h_ref[...], w3a_ref[...], (((1,), (1,)), ((), ())),
                           preferred_element_type=jnp.float32)
           + lax.dot_general(agg, w3b_ref[...], (((1,), (1,)), ((), ())),
                             preferred_element_type=jnp.float32)
           + b3_ref[...])
    out_ref[...] = jnp.maximum(pre, 0.0)


def _proj(h, wstack, bstack):
    nb = N_NODES // _NODE_BLK
    return pl.pallas_call(
        _proj_body,
        grid=(2 * nb,),
        in_specs=[pl.BlockSpec((_NODE_BLK, D), lambda i: (i % nb, 0)),
                  pl.BlockSpec((1, D, D), lambda i: (i // nb, 0, 0)),
                  pl.BlockSpec((1, 1, D), lambda i: (i // nb, 0, 0))],
        out_specs=pl.BlockSpec((_NODE_BLK, D), lambda i: (i, 0)),
        out_shape=jax.ShapeDtypeStruct((2 * N_NODES, D), jnp.float32),
    )(h, wstack, bstack)


def _msg(spk, e, w1cp, w2p, b2r):
    ne = e.shape[0]
    hblk = _EDGE_BLK // 2
    nb = ne // _EDGE_BLK
    bpt = (ne // 64) // hblk
    full = lambda i: (0, 0)
    lo_blk = lambda i: (2 * bpt * (i // bpt) + (i % bpt), 0)
    hi_blk = lambda i: (2 * bpt * (i // bpt) + (i % bpt) + bpt, 0)
    return pl.pallas_call(
        _msg_body,
        grid=(nb,),
        in_specs=[pl.BlockSpec((hblk, D), lambda i: (i, 0)),
                  pl.BlockSpec((hblk, ED), lo_blk),
                  pl.BlockSpec((hblk, ED), hi_blk),
                  pl.BlockSpec((D, ED), full),
                  pl.BlockSpec((D, D), full),
                  pl.BlockSpec((1, D), full)],
        out_specs=[pl.BlockSpec((hblk, D), lambda i: (i, 0)),
                   pl.BlockSpec((hblk, D), lambda i: (i, 0))],
        out_shape=[jax.ShapeDtypeStruct((ne // 2, D), jnp.float32),
                   jax.ShapeDtypeStruct((ne // 2, D), jnp.float32)],
    )(spk, e, e, w1cp, w2p, b2r)


def _update(h, parts, w3a, w3b, b3r):
    nb = N_NODES // _NODE_BLK
    full = lambda i: (0, 0)
    nparts = parts.shape[0]
    return pl.pallas_call(
        _update_body,
        grid=(nb,),
        in_specs=[pl.BlockSpec((_NODE_BLK, D), lambda i: (i, 0)),
                  pl.BlockSpec((nparts, _NODE_BLK, D), lambda i: (0, i, 0)),
                  pl.BlockSpec((D, D), full),
                  pl.BlockSpec((D, D), full),
                  pl.BlockSpec((1, D), full)],
        out_specs=pl.BlockSpec((_NODE_BLK, D), lambda i: (i, 0)),
        out_shape=jax.ShapeDtypeStruct((N_NODES, D), jnp.float32),
    )(h, parts, w3a, w3b, b3r)


def _sc_info():
    try:
        info = plsc.get_sparse_core_info()
        return info.num_cores, info.num_subcores
    except Exception:
        return 2, 16


_GNB = 2


@functools.cache
def _make_gather(ne):
    nc, ns = _sc_info()
    nw = nc * ns
    epw = ne // nw
    opw = epw // 2
    orows = _CH // 2
    nch = opw // orows
    nrounds = nch // _GNB
    mesh = plsc.VectorSubcoreMesh(core_axis_name="c", subcore_axis_name="s")

    @functools.partial(
        pl.kernel, mesh=mesh,
        out_type=jax.ShapeDtypeStruct((ne // 2, D), jnp.int32),
        scratch_types=[pltpu.VMEM((2 * epw,), jnp.int32),
                       pltpu.VMEM((2, 2, _GNB, _CH, D), jnp.float32),
                       pltpu.VMEM((2, _GNB, orows, D), jnp.int32),
                       pltpu.SemaphoreType.DMA((2, _GNB)),
                       pltpu.SemaphoreType.DMA((2, _GNB))])
    def gather_k(t_hbm, comb_hbm, spk_hbm, cidx, gbuf, obuf, gsem, wsem):
        wid = lax.axis_index("s") * nc + lax.axis_index("c")
        base = wid * epw
        obase = wid * opw
        pltpu.sync_copy(comb_hbm.at[pl.ds(2 * base, 2 * epw)], cidx)
        half = jnp.int32(32768)
        topmask = jnp.int32(-65536)

        def fire_gather(p, b, g):
            for hh in range(2):
                ioff = 2 * (hh * opw + g * orows)
                pltpu.async_copy(t_hbm.at[cidx.at[pl.ds(ioff, _CH)]],
                                 gbuf.at[hh, p, b], gsem.at[p, b])

        def wait_gather(p, b):
            for hh in range(2):
                pltpu.make_async_copy(t_hbm.at[pl.ds(0, _CH)],
                                      gbuf.at[hh, p, b], gsem.at[p, b]).wait()

        def fire_write(p, b, g):
            pltpu.async_copy(obuf.at[p, b],
                             spk_hbm.at[pl.ds(obase + g * orows, orows)],
                             wsem.at[p, b])

        def wait_write(p, b):
            pltpu.make_async_copy(obuf.at[p, b],
                                  spk_hbm.at[pl.ds(obase, orows)],
                                  wsem.at[p, b]).wait()

        def pack_chunk(p, b):
            def row(j, carry):
                jb = j + orows
                for hh in range(2):
                    for cc in range(4):
                        c = cc * 32
                        va0 = (gbuf[hh, p, b, j, pl.ds(c, 16)]
                               + gbuf[hh, p, b, jb, pl.ds(c, 16)])
                        va1 = (gbuf[hh, p, b, j, pl.ds(c + 16, 16)]
                               + gbuf[hh, p, b, jb, pl.ds(c + 16, 16)])
                        lo = lax.shift_right_logical(
                            lax.bitcast_convert_type(va0, jnp.int32) + half, 16)
                        hi = jnp.bitwise_and(
                            lax.bitcast_convert_type(va1, jnp.int32) + half,
                            topmask)
                        obuf[p, b, j, pl.ds(hh * 64 + cc * 16, 16)] = (
                            jnp.bitwise_or(lo, hi))
                return carry
            lax.fori_loop(0, orows, row, 0)

        for b in range(_GNB):
            fire_gather(0, b, b)

        def super_round(r2, carry):
            for p in range(2):
                r = 2 * r2 + p
                for b in range(_GNB):
                    g = r * _GNB + b
                    wait_gather(p, b)

                    @pl.when(r >= 2)
                    def _():
                        wait_write(p, b)

                    pack_chunk(p, b)
                    fire_write(p, b, g)

                    @pl.when(r < nrounds - 1)
                    def _():
                        fire_gather(1 - p, b, g + _GNB)
            return carry

        lax.fori_loop(0, nrounds // 2, super_round, 0)
        for p in range(2):
            for b in range(_GNB):
                wait_write(p, b)
        for g in range(nrounds * _GNB, nch):
            fire_gather(0, 0, g)
            wait_gather(0, 0)
            pack_chunk(0, 0)
            pltpu.sync_copy(obuf.at[0, 0],
                            spk_hbm.at[pl.ds(obase + g * orows, orows)])

    return gather_k


@functools.cache
def _make_scatter(nseg, seg):
    nc, ns = _sc_info()
    nw = nc * ns
    spw = seg // nw
    hpw = spw // 2
    sch = 40
    nch = hpw // sch
    rpt = -(-N_NODES // (ns * _CH)) * _CH
    npad = rpt * ns
    spt = rpt // _CH
    mesh = plsc.VectorSubcoreMesh(core_axis_name="c", subcore_axis_name="s")

    snb = 2
    nrounds = nch // snb

    @functools.partial(
        pl.kernel, mesh=mesh,
        out_type=jax.ShapeDtypeStruct((nc, npad, D), jnp.float32),
        scratch_types=[pltpu.VMEM((2, snb, sch), jnp.int32),
                       pltpu.VMEM((2, snb, sch, D), jnp.float32),
                       pltpu.VMEM((_CH, D), jnp.float32),
                       pltpu.SemaphoreType.DMA((2, snb)),
                       pltpu.SemaphoreType.DMA((2, snb)),
                       pltpu.VMEM_SHARED((npad, D), jnp.float32)])
    def scatter_k(*refs):
        msgs = refs[:2 * nseg]
        (dst_hbm, zeros_hbm, out_hbm,
         didx, mbuf, zbuf, lsem, ssem, acc) = refs[2 * nseg:]
        c = lax.axis_index("c")
        s = lax.axis_index("s")
        wid = s * nc + c
        base = wid * hpw

        pltpu.sync_copy(zeros_hbm, zbuf)
        for k in range(spt):
            pltpu.sync_copy(zbuf, acc.at[pl.ds(s * rpt + k * _CH, _CH)])
        plsc.subcore_barrier()

        def run_stream(msg_hbm, dbase):
            def fire_load(p, b, g):
                off = g * sch
                pltpu.async_copy(dst_hbm.at[pl.ds(dbase + off, sch)],
                                 didx.at[p, b], lsem.at[p, b])
                pltpu.async_copy(msg_hbm.at[pl.ds(base + off, sch)],
                                 mbuf.at[p, b], lsem.at[p, b])

            def wait_load(p, b):
                pltpu.make_async_copy(dst_hbm.at[pl.ds(dbase, sch)],
                                      didx.at[p, b], lsem.at[p, b]).wait()
                pltpu.make_async_copy(msg_hbm.at[pl.ds(base, sch)],
                                      mbuf.at[p, b], lsem.at[p, b]).wait()

            def fire_scatter(p, b):
                pltpu.async_copy(mbuf.at[p, b], acc.at[didx.at[p, b]],
                                 ssem.at[p, b], add=True)

            def wait_scatter(p, b):
                pltpu.make_async_copy(mbuf.at[p, b], acc.at[didx.at[p, b]],
                                      ssem.at[p, b]).wait()

            for b in range(snb):
                fire_load(0, b, b)

            def super_round(r2, carry):
                for p in range(2):
                    r = 2 * r2 + p
                    for b in range(snb):
                        g = r * snb + b
                        wait_load(p, b)
                        fire_scatter(p, b)

                        @pl.when(r >= 1)
                        def _():
                            wait_scatter(1 - p, b)

                        @pl.when(r < nrounds - 1)
                        def _():
                            fire_load(1 - p, b, g + snb)
                return carry

            lax.fori_loop(0, nrounds // 2, super_round, 0)
            for b in range(snb):
                wait_scatter((nrounds - 1) % 2, b)
            for g in range(nrounds * snb, nch):
                off = g * sch
                pltpu.sync_copy(dst_hbm.at[pl.ds(dbase + off, sch)],
                                didx.at[0, 0])
                pltpu.sync_copy(msg_hbm.at[pl.ds(base + off, sch)],
                                mbuf.at[0, 0])
                pltpu.sync_copy(mbuf.at[0, 0], acc.at[didx.at[0, 0]], add=True)

        for k in range(nseg):
            run_stream(msgs[2 * k], k * seg + wid * spw)
            run_stream(msgs[2 * k + 1], k * seg + wid * spw + hpw)

        plsc.subcore_barrier()
        for k in range(spt):
            pltpu.sync_copy(acc.at[pl.ds(s * rpt + k * _CH, _CH)], zbuf)
            pltpu.sync_copy(zbuf,
                            out_hbm.at[c, pl.ds(s * rpt + k * _CH, _CH)])

    return scatter_k


_PI = [32 * (t // 16) + (t % 16) for t in range(64)] + \
      [32 * (t // 16) + 16 + (t % 16) for t in range(64)]


def kernel(h, edge_index, e, W1, b1, W2, b2, W3, b3):
    src = edge_index[0]
    dst = edge_index[1]
    w1a = W1[:, :D]
    w1b = W1[:, D:2 * D]
    w1c = W1[:, 2 * D:]
    w3a = W3[:, :D]
    w3b = W3[:, D:]
    b1r = b1.reshape(1, D)
    b2r = b2.reshape(1, D)
    b3r = b3.reshape(1, D)
    pi = jnp.asarray(_PI, dtype=jnp.int32)
    w1cp = w1c[pi, :]
    w2p = W2[:, pi]

    wstack = jnp.stack([w1a, w1b])
    bstack = jnp.stack([b1r, jnp.zeros_like(b1r)])
    t = _proj(h, wstack, bstack)
    comb = jnp.stack([src.reshape(-1, 40),
                      dst.reshape(-1, 40) + N_NODES], axis=1).reshape(-1)
    nseg = 5
    seg = N_EDGES // nseg
    gather = _make_gather(seg)
    msgs = []
    for k in range(nseg):
        sl = slice(k * seg, (k + 1) * seg)
        spk = gather(t, comb[2 * k * seg:2 * (k + 1) * seg])
        me, mo = _msg(spk, e[sl], w1cp, w2p, b2r)
        msgs += [me, mo]
    zrows = jnp.zeros((_CH, D), jnp.float32)
    parts = _make_scatter(nseg, seg)(*msgs, dst, zrows)
    return _update(h, parts, w3a, w3b, b3r)


# speedup vs baseline: 1.1160x; 1.1160x over previous
import functools

import jax
import jax.numpy as jnp
from jax import lax
from jax.experimental import pallas as pl
from jax.experimental.pallas import tpu as pltpu
from jax.experimental.pallas import tpu_sc as plsc

N_NODES = 10000
N_EDGES = 320000
D = 128
ED = 16

_NODE_BLK = 2000
_EDGE_BLK = 2000
_CH = 80


def _proj_body(h_ref, w1a_ref, w1b_ref, b1_ref, a_ref, b_ref):
    h = h_ref[...]
    a_ref[...] = lax.dot_general(h, w1a_ref[...], (((1,), (1,)), ((), ())),
                                 preferred_element_type=jnp.float32) + b1_ref[...]
    b_ref[...] = lax.dot_general(h, w1b_ref[...], (((1,), (1,)), ((), ())),
                                 preferred_element_type=jnp.float32)


def _msg_body(spk_ref, ee_ref, eo_ref, w1cp_ref, w2p_ref, b2_ref,
              me_ref, mo_ref):
    w = spk_ref[...]
    lo = lax.bitcast_convert_type(lax.shift_left(w, 16), jnp.float32)
    hi = lax.bitcast_convert_type(
        jnp.bitwise_and(w, jnp.int32(-65536)), jnp.float32)
    even_pre = jnp.concatenate([lo[:, :D // 2], hi[:, :D // 2]], axis=1)
    odd_pre = jnp.concatenate([lo[:, D // 2:], hi[:, D // 2:]], axis=1)
    dn = (((1,), (1,)), ((), ()))
    ce = lax.dot_general(ee_ref[...], w1cp_ref[...], dn,
                         preferred_element_type=jnp.float32)
    co = lax.dot_general(eo_ref[...], w1cp_ref[...], dn,
                         preferred_element_type=jnp.float32)
    he = jnp.maximum(even_pre + ce, 0.0)
    ho = jnp.maximum(odd_pre + co, 0.0)
    me_ref[...] = lax.dot_general(he, w2p_ref[...], dn,
                                  preferred_element_type=jnp.float32) + b2_ref[...]
    mo_ref[...] = lax.dot_general(ho, w2p_ref[...], dn,
                                  preferred_element_type=jnp.float32) + b2_ref[...]


def _update_body(h_ref, p_ref, w3a_ref, w3b_ref, b3_ref, out_ref):
    agg = p_ref[0] + p_ref[1]
    pre = (lax.dot_general(h_ref[...], w3a_ref[...], (((1,), (1,)), ((), ())),
                           preferred_element_type=jnp.float32)
           + lax.dot_general(agg, w3b_ref[...], (((1,), (1,)), ((), ())),
                             preferred_element_type=jnp.float32)
           + b3_ref[...])
    out_ref[...] = jnp.maximum(pre, 0.0)


def _proj(h, w1a, w1b, b1r):
    nb = N_NODES // _NODE_BLK
    full = lambda i: (0, 0)
    return pl.pallas_call(
        _proj_body,
        grid=(nb,),
        in_specs=[pl.BlockSpec((_NODE_BLK, D), lambda i: (i, 0)),
                  pl.BlockSpec((D, D), full),
                  pl.BlockSpec((D, D), full),
                  pl.BlockSpec((1, D), full)],
        out_specs=[pl.BlockSpec((_NODE_BLK, D), lambda i: (i, 0)),
                   pl.BlockSpec((_NODE_BLK, D), lambda i: (i, 0))],
        out_shape=[jax.ShapeDtypeStruct((N_NODES, D), jnp.float32),
                   jax.ShapeDtypeStruct((N_NODES, D), jnp.float32)],
    )(h, w1a, w1b, b1r)


def _msg(spk, e, w1cp, w2p, b2r):
    ne = e.shape[0]
    hblk = _EDGE_BLK // 2
    nb = ne // _EDGE_BLK
    bpt = (ne // 64) // hblk
    full = lambda i: (0, 0)
    lo_blk = lambda i: (2 * bpt * (i // bpt) + (i % bpt), 0)
    hi_blk = lambda i: (2 * bpt * (i // bpt) + (i % bpt) + bpt, 0)
    return pl.pallas_call(
        _msg_body,
        grid=(nb,),
        in_specs=[pl.BlockSpec((hblk, D), lambda i: (i, 0)),
                  pl.BlockSpec((hblk, ED), lo_blk),
                  pl.BlockSpec((hblk, ED), hi_blk),
                  pl.BlockSpec((D, ED), full),
                  pl.BlockSpec((D, D), full),
                  pl.BlockSpec((1, D), full)],
        out_specs=[pl.BlockSpec((hblk, D), lambda i: (i, 0)),
                   pl.BlockSpec((hblk, D), lambda i: (i, 0))],
        out_shape=[jax.ShapeDtypeStruct((ne // 2, D), jnp.float32),
                   jax.ShapeDtypeStruct((ne // 2, D), jnp.float32)],
    )(spk, e, e, w1cp, w2p, b2r)


def _update(h, parts, w3a, w3b, b3r):
    nb = N_NODES // _NODE_BLK
    full = lambda i: (0, 0)
    nparts = parts.shape[0]
    return pl.pallas_call(
        _update_body,
        grid=(nb,),
        in_specs=[pl.BlockSpec((_NODE_BLK, D), lambda i: (i, 0)),
                  pl.BlockSpec((nparts, _NODE_BLK, D), lambda i: (0, i, 0)),
                  pl.BlockSpec((D, D), full),
                  pl.BlockSpec((D, D), full),
                  pl.BlockSpec((1, D), full)],
        out_specs=pl.BlockSpec((_NODE_BLK, D), lambda i: (i, 0)),
        out_shape=jax.ShapeDtypeStruct((N_NODES, D), jnp.float32),
    )(h, parts, w3a, w3b, b3r)


def _sc_info():
    try:
        info = plsc.get_sparse_core_info()
        return info.num_cores, info.num_subcores
    except Exception:
        return 2, 16


_GNB = 2


@functools.cache
def _make_gather(ne):
    nc, ns = _sc_info()
    nw = nc * ns
    epw = ne // nw
    opw = epw // 2
    orows = _CH // 2
    nch = opw // orows
    nrounds = nch // _GNB
    mesh = plsc.VectorSubcoreMesh(core_axis_name="c", subcore_axis_name="s")

    @functools.partial(
        pl.kernel, mesh=mesh,
        out_type=jax.ShapeDtypeStruct((ne // 2, D), jnp.int32),
        scratch_types=[pltpu.VMEM((epw,), jnp.int32),
                       pltpu.VMEM((epw,), jnp.int32),
                       pltpu.VMEM((2, 2, _GNB, orows, D), jnp.float32),
                       pltpu.VMEM((2, 2, _GNB, orows, D), jnp.float32),
                       pltpu.VMEM((2, _GNB, orows, D), jnp.int32),
                       pltpu.SemaphoreType.DMA((2, _GNB)),
                       pltpu.SemaphoreType.DMA((2, _GNB))])
    def gather_k(a_hbm, b_hbm, src_hbm, dst_hbm, spk_hbm,
                 sidx, didx, abuf, bbuf, obuf, gsem, wsem):
        wid = lax.axis_index("s") * nc + lax.axis_index("c")
        base = wid * epw
        obase = wid * opw
        pltpu.sync_copy(src_hbm.at[pl.ds(base, epw)], sidx)
        pltpu.sync_copy(dst_hbm.at[pl.ds(base, epw)], didx)
        half = jnp.int32(32768)
        topmask = jnp.int32(-65536)

        def fire_gather(p, b, g):
            for hh in range(2):
                ioff = hh * opw + g * orows
                pltpu.async_copy(a_hbm.at[sidx.at[pl.ds(ioff, orows)]],
                                 abuf.at[hh, p, b], gsem.at[p, b])
                pltpu.async_copy(b_hbm.at[didx.at[pl.ds(ioff, orows)]],
                                 bbuf.at[hh, p, b], gsem.at[p, b])

        def wait_gather(p, b):
            for hh in range(2):
                pltpu.make_async_copy(a_hbm.at[pl.ds(0, orows)],
                                      abuf.at[hh, p, b], gsem.at[p, b]).wait()
                pltpu.make_async_copy(b_hbm.at[pl.ds(0, orows)],
                                      bbuf.at[hh, p, b], gsem.at[p, b]).wait()

        def fire_write(p, b, g):
            pltpu.async_copy(obuf.at[p, b],
                             spk_hbm.at[pl.ds(obase + g * orows, orows)],
                             wsem.at[p, b])

        def wait_write(p, b):
            pltpu.make_async_copy(obuf.at[p, b],
                                  spk_hbm.at[pl.ds(obase, orows)],
                                  wsem.at[p, b]).wait()

        def pack_chunk(p, b):
            def row(j, carry):
                for hh in range(2):
                    for cc in range(4):
                        c = cc * 32
                        va0 = (abuf[hh, p, b, j, pl.ds(c, 16)]
                               + bbuf[hh, p, b, j, pl.ds(c, 16)])
                        va1 = (abuf[hh, p, b, j, pl.ds(c + 16, 16)]
                               + bbuf[hh, p, b, j, pl.ds(c + 16, 16)])
                        lo = lax.shift_right_logical(
                            lax.bitcast_convert_type(va0, jnp.int32) + half, 16)
                        hi = jnp.bitwise_and(
                            lax.bitcast_convert_type(va1, jnp.int32) + half,
                            topmask)
                        obuf[p, b, j, pl.ds(hh * 64 + cc * 16, 16)] = (
                            jnp.bitwise_or(lo, hi))
                return carry
            lax.fori_loop(0, orows, row, 0)

        for b in range(_GNB):
            fire_gather(0, b, b)

        def super_round(r2, carry):
            for p in range(2):
                r = 2 * r2 + p
                for b in range(_GNB):
                    g = r * _GNB + b
                    wait_gather(p, b)

                    @pl.when(r >= 2)
                    def _():
                        wait_write(p, b)

                    pack_chunk(p, b)
                    fire_write(p, b, g)

                    @pl.when(r < nrounds - 1)
                    def _():
                        fire_gather(1 - p, b, g + _GNB)
            return carry

        lax.fori_loop(0, nrounds // 2, super_round, 0)
        for p in range(2):
            for b in range(_GNB):
                wait_write(p, b)
        for g in range(nrounds * _GNB, nch):
            fire_gather(0, 0, g)
            wait_gather(0, 0)
            pack_chunk(0, 0)
            pltpu.sync_copy(obuf.at[0, 0],
                            spk_hbm.at[pl.ds(obase + g * orows, orows)])

    return gather_k


@functools.cache
def _make_scatter(nseg, seg):
    nc, ns = _sc_info()
    nw = nc * ns
    spw = seg // nw
    hpw = spw // 2
    sch = 40
    nch = hpw // sch
    rpt = -(-N_NODES // (ns * _CH)) * _CH
    npad = rpt * ns
    spt = rpt // _CH
    mesh = plsc.VectorSubcoreMesh(core_axis_name="c", subcore_axis_name="s")

    snb = 2
    nrounds = nch // snb

    @functools.partial(
        pl.kernel, mesh=mesh,
        out_type=jax.ShapeDtypeStruct((nc, npad, D), jnp.float32),
        scratch_types=[pltpu.VMEM((2, snb, sch), jnp.int32),
                       pltpu.VMEM((2, snb, sch, D), jnp.float32),
                       pltpu.VMEM((_CH, D), jnp.float32),
                       pltpu.SemaphoreType.DMA((2, snb)),
                       pltpu.SemaphoreType.DMA((2, snb)),
                       pltpu.VMEM_SHARED((npad, D), jnp.float32)])
    def scatter_k(*refs):
        msgs = refs[:2 * nseg]
        (dst_hbm, zeros_hbm, out_hbm,
         didx, mbuf, zbuf, lsem, ssem, acc) = refs[2 * nseg:]
        c = lax.axis_index("c")
        s = lax.axis_index("s")
        wid = s * nc + c
        base = wid * hpw

        pltpu.sync_copy(zeros_hbm, zbuf)
        for k in range(spt):
            pltpu.sync_copy(zbuf, acc.at[pl.ds(s * rpt + k * _CH, _CH)])
        plsc.subcore_barrier()

        def run_stream(msg_hbm, dbase):
            def fire_load(p, b, g):
                off = g * sch
                pltpu.async_copy(dst_hbm.at[pl.ds(dbase + off, sch)],
                                 didx.at[p, b], lsem.at[p, b])
                pltpu.async_copy(msg_hbm.at[pl.ds(base + off, sch)],
                                 mbuf.at[p, b], lsem.at[p, b])

            def wait_load(p, b):
                pltpu.make_async_copy(dst_hbm.at[pl.ds(dbase, sch)],
                                      didx.at[p, b], lsem.at[p, b]).wait()
                pltpu.make_async_copy(msg_hbm.at[pl.ds(base, sch)],
                                      mbuf.at[p, b], lsem.at[p, b]).wait()

            def fire_scatter(p, b):
                pltpu.async_copy(mbuf.at[p, b], acc.at[didx.at[p, b]],
                                 ssem.at[p, b], add=True)

            def wait_scatter(p, b):
                pltpu.make_async_copy(mbuf.at[p, b], acc.at[didx.at[p, b]],
                                      ssem.at[p, b]).wait()

            for b in range(snb):
                fire_load(0, b, b)

            def super_round(r2, carry):
                for p in range(2):
                    r = 2 * r2 + p
                    for b in range(snb):
                        g = r * snb + b
                        wait_load(p, b)
                        fire_scatter(p, b)

                        @pl.when(r >= 1)
                        def _():
                            wait_scatter(1 - p, b)

                        @pl.when(r < nrounds - 1)
                        def _():
                            fire_load(1 - p, b, g + snb)
                return carry

            lax.fori_loop(0, nrounds // 2, super_round, 0)
            for b in range(snb):
                wait_scatter((nrounds - 1) % 2, b)
            for g in range(nrounds * snb, nch):
                off = g * sch
                pltpu.sync_copy(dst_hbm.at[pl.ds(dbase + off, sch)],
                                didx.at[0, 0])
                pltpu.sync_copy(msg_hbm.at[pl.ds(base + off, sch)],
                                mbuf.at[0, 0])
                pltpu.sync_copy(mbuf.at[0, 0], acc.at[didx.at[0, 0]], add=True)

        for k in range(nseg):
            run_stream(msgs[2 * k], k * seg + wid * spw)
            run_stream(msgs[2 * k + 1], k * seg + wid * spw + hpw)

        plsc.subcore_barrier()
        for k in range(spt):
            pltpu.sync_copy(acc.at[pl.ds(s * rpt + k * _CH, _CH)], zbuf)
            pltpu.sync_copy(zbuf,
                            out_hbm.at[c, pl.ds(s * rpt + k * _CH, _CH)])

    return scatter_k


_PI = [32 * (t // 16) + (t % 16) for t in range(64)] + \
      [32 * (t // 16) + 16 + (t % 16) for t in range(64)]


def kernel(h, edge_index, e, W1, b1, W2, b2, W3, b3):
    src = edge_index[0]
    dst = edge_index[1]
    w1a = W1[:, :D]
    w1b = W1[:, D:2 * D]
    w1c = W1[:, 2 * D:]
    w3a = W3[:, :D]
    w3b = W3[:, D:]
    b1r = b1.reshape(1, D)
    b2r = b2.reshape(1, D)
    b3r = b3.reshape(1, D)
    pi = jnp.asarray(_PI, dtype=jnp.int32)
    w1cp = w1c[pi, :]
    w2p = W2[:, pi]

    a, b = _proj(h, w1a, w1b, b1r)
    nseg = 5
    seg = N_EDGES // nseg
    gather = _make_gather(seg)
    msgs = []
    for k in range(nseg):
        sl = slice(k * seg, (k + 1) * seg)
        spk = gather(a, b, src[sl], dst[sl])
        me, mo = _msg(spk, e[sl], w1cp, w2p, b2r)
        msgs += [me, mo]
    zrows = jnp.zeros((_CH, D), jnp.float32)
    parts = _make_scatter(nseg, seg)(*msgs, dst, zrows)
    return _update(h, parts, w3a, w3b, b3r)
